# baseline (speedup 1.0000x reference)
import functools

import jax
import jax.numpy as jnp
from jax import lax
from jax.experimental import pallas as pl
from jax.experimental.pallas import tpu as pltpu
from jax.experimental.pallas import tpu_sc as plsc

VOCAB = 1000000
D = 64
BATCH = 4096
HIST = 50
NC = 2
NS = 16
NW = NC * NS
CHUNK = BATCH // NW
L = 16
PADD = 80
NBUF = 5
N_GROUPS = HIST // NBUF


def _emb_body(idx_hbm, table_hbm, out_hbm, idx_v, rows_v, *sems):
    gsems, osems = sems[:NBUF], sems[NBUF:]
    wid = lax.axis_index("s") * NC + lax.axis_index("c")
    b0 = wid * CHUNK
    pltpu.sync_copy(idx_hbm.at[:, pl.ds(b0, CHUNK)], idx_v)

    def g_desc(k, b):
        return pltpu.make_async_copy(
            table_hbm.at[idx_v.at[k]], rows_v.at[b], gsems[b])

    def o_desc(k, b):
        return pltpu.make_async_copy(
            rows_v.at[b].at[:, pl.ds(0, D)],
            out_hbm.at[pl.ds(b0, CHUNK), k], osems[b])

    for b in range(NBUF):
        g_desc(b, b).start()

    def group(g, carry):
        for b in range(NBUF):
            k = g * NBUF + b
            g_desc(k, b).wait()
            o_desc(k, b).start()
            o_desc(k, b).wait()
            g_desc(k + NBUF, b).start()
        return carry

    lax.fori_loop(0, N_GROUPS - 1, group, 0)

    for b in range(NBUF):
        k = (N_GROUPS - 1) * NBUF + b
        g_desc(k, b).wait()
        o_desc(k, b).start()
    for b in range(NBUF):
        k = (N_GROUPS - 1) * NBUF + b
        o_desc(k, b).wait()


@jax.jit
def kernel(x, table):
    xt = x.T.astype(jnp.int32)
    tp = jnp.pad(table, ((0, 0), (0, PADD - D)))
    run = pl.kernel(
        _emb_body,
        mesh=plsc.VectorSubcoreMesh(core_axis_name="c", subcore_axis_name="s"),
        out_type=jax.ShapeDtypeStruct((BATCH, HIST, D), jnp.float32),
        scratch_types=[
            pltpu.VMEM((HIST, CHUNK), jnp.int32),
            pltpu.VMEM((NBUF, CHUNK, PADD), jnp.float32),
        ] + [pltpu.SemaphoreType.DMA] * (2 * NBUF),
        compiler_params=pltpu.CompilerParams(use_tc_tiling_on_sc=False),
    )
    return run(xt, tp)


# speedup vs baseline: 1.7758x; 1.7758x over previous
import functools

import jax
import jax.numpy as jnp
from jax import lax
from jax.experimental import pallas as pl
from jax.experimental.pallas import tpu as pltpu
from jax.experimental.pallas import tpu_sc as plsc

VOCAB = 1000000
D = 64
BATCH = 4096
HIST = 50
NC = 2
NS = 16
NW = NC * NS
CHUNK = BATCH // NW
L = 16
NBUF = 5
N_GROUPS = HIST // NBUF


def _emb_body(idx_hbm, table_hbm, out_hbm, idx_v, rows_v, *sems):
    gsems, osems = sems[:NBUF], sems[NBUF:]
    wid = lax.axis_index("s") * NC + lax.axis_index("c")
    b0 = wid * CHUNK
    pltpu.sync_copy(idx_hbm.at[:, pl.ds(b0, CHUNK)], idx_v)

    @plsc.parallel_loop(0, HIST)
    def _double_row(r):
        for j in range(CHUNK // L):
            idx_v[r, pl.ds(j * L, L)] = idx_v[r, pl.ds(j * L, L)] * 2

    def g_desc(k, b):
        return pltpu.make_async_copy(
            table_hbm.at[idx_v.at[k]], rows_v.at[b], gsems[b])

    def o_desc(k, b):
        return pltpu.make_async_copy(
            rows_v.at[b], out_hbm.at[pl.ds(b0, CHUNK), k], osems[b])

    for b in range(NBUF):
        g_desc(b, b).start()

    def group(g, carry):
        for b in range(NBUF):
            k = g * NBUF + b
            g_desc(k, b).wait()
            o_desc(k, b).start()
            o_desc(k, b).wait()
            g_desc(k + NBUF, b).start()
        return carry

    lax.fori_loop(0, N_GROUPS - 1, group, 0)

    for b in range(NBUF):
        k = (N_GROUPS - 1) * NBUF + b
        g_desc(k, b).wait()
        o_desc(k, b).start()
    for b in range(NBUF):
        k = (N_GROUPS - 1) * NBUF + b
        o_desc(k, b).wait()


@jax.jit
def kernel(x, table):
    xt = x.T.astype(jnp.int32)
    tp = jnp.pad(table, ((0, 0), (0, D))).reshape(2 * VOCAB, D)
    run = pl.kernel(
        _emb_body,
        mesh=plsc.VectorSubcoreMesh(core_axis_name="c", subcore_axis_name="s"),
        out_type=jax.ShapeDtypeStruct((BATCH, HIST, D), jnp.float32),
        scratch_types=[
            pltpu.VMEM((HIST, CHUNK), jnp.int32),
            pltpu.VMEM((NBUF, CHUNK, D), jnp.float32),
        ] + [pltpu.SemaphoreType.DMA] * (2 * NBUF),
        compiler_params=pltpu.CompilerParams(use_tc_tiling_on_sc=False),
    )
    return run(xt, tp)
